# baseline (speedup 1.0000x reference)
import functools

import jax
import jax.numpy as jnp
from jax import lax
from jax.experimental import pallas as pl
from jax.experimental.pallas import tpu as pltpu
from jax.experimental.pallas import tpu_sc as plsc

N = 10000
D = 128
E = 320000

NC = 2
NS = 16
NW = NC * NS
EPW = E // NW
CHUNK = 40
NCHUNK = EPW // CHUNK
NPAD = 10240
AGG_PER_SUB = NPAD // NS
DEG_PER_SUB = NPAD // NS

ROWS_BLK = 1024


def _sc_aggregate(h, src, dst):
    mesh = plsc.VectorSubcoreMesh(
        core_axis_name="c", subcore_axis_name="s", num_cores=NC, num_subcores=NS
    )

    @functools.partial(
        pl.kernel,
        out_type=(
            jax.ShapeDtypeStruct((NC, NPAD, D), jnp.float32),
            jax.ShapeDtypeStruct((NC, NPAD), jnp.float32),
        ),
        mesh=mesh,
        scratch_types=(
            pltpu.VMEM_SHARED((NPAD, D), jnp.float32),
            pltpu.VMEM_SHARED((NPAD,), jnp.float32),
            pltpu.VMEM((CHUNK,), jnp.int32),
            pltpu.VMEM((CHUNK,), jnp.int32),
            pltpu.VMEM((CHUNK,), jnp.int32),
            pltpu.VMEM((CHUNK,), jnp.int32),
            pltpu.VMEM((CHUNK, D), jnp.float32),
            pltpu.VMEM((CHUNK, D), jnp.float32),
            pltpu.VMEM((CHUNK,), jnp.float32),
            pltpu.VMEM((DEG_PER_SUB,), jnp.float32),
            pltpu.SemaphoreType.DMA,
            pltpu.SemaphoreType.DMA,
        ),
    )
    def k(h_hbm, src_hbm, dst_hbm,
          agg_out, deg_out, agg_sh, deg_sh, sa_s, sa_d, sb_s, sb_d,
          rows0, rows1, ones_v, zcol, g0, g1):
        c = lax.axis_index("c")
        s = lax.axis_index("s")
        wid = c * NS + s

        o16 = jnp.ones((16,), jnp.float32)
        z16 = jnp.zeros((16,), jnp.float32)
        for t in range(CHUNK // 16):
            ones_v[pl.ds(16 * t, 16)] = o16

        def zrow(i, carry):
            r = i // (D // 16)
            q = i - r * (D // 16)
            rows0[r, pl.ds(q * 16, 16)] = z16
            return carry

        lax.fori_loop(0, CHUNK * (D // 16), zrow, 0)

        def zvec(i, carry):
            zcol[pl.ds(i * 16, 16)] = z16
            return carry

        lax.fori_loop(0, DEG_PER_SUB // 16, zvec, 0)

        for t in range(AGG_PER_SUB // CHUNK):
            pltpu.sync_copy(
                rows0, agg_sh.at[pl.ds(s * AGG_PER_SUB + t * CHUNK, CHUNK)]
            )

        pltpu.sync_copy(zcol, deg_sh.at[pl.ds(s * DEG_PER_SUB, DEG_PER_SUB)])
        plsc.subcore_barrier()

        def idx_fetch(ci, bs, bd):
            pltpu.sync_copy(src_hbm.at[wid, ci], bs)
            pltpu.sync_copy(dst_hbm.at[wid, ci], bd)

        def g_fire(sidx, rows, sem):
            pltpu.async_copy(h_hbm.at[sidx], rows, sem)

        def g_wait(rows, sem):
            pltpu.make_async_copy(h_hbm.at[sa_s], rows, sem).wait()

        def scat(didx, rows):
            pltpu.sync_copy(rows, agg_sh.at[didx], add=True)
            pltpu.sync_copy(ones_v, deg_sh.at[didx], add=True)

        idx_fetch(0, sa_s, sa_d)
        g_fire(sa_s, rows0, g0)

        def body(i, carry):
            c0 = 2 * i
            idx_fetch(c0 + 1, sb_s, sb_d)
            g_fire(sb_s, rows1, g1)
            g_wait(rows0, g0)
            scat(sa_d, rows0)
            idx_fetch(c0 + 2, sa_s, sa_d)
            g_fire(sa_s, rows0, g0)
            g_wait(rows1, g1)
            scat(sb_d, rows1)
            return carry

        lax.fori_loop(0, (NCHUNK - 2) // 2, body, 0)
        idx_fetch(NCHUNK - 1, sb_s, sb_d)
        g_fire(sb_s, rows1, g1)
        g_wait(rows0, g0)
        scat(sa_d, rows0)
        g_wait(rows1, g1)
        scat(sb_d, rows1)
        plsc.subcore_barrier()

        pltpu.sync_copy(
            agg_sh.at[pl.ds(s * AGG_PER_SUB, AGG_PER_SUB)],
            agg_out.at[c, pl.ds(s * AGG_PER_SUB, AGG_PER_SUB)],
        )
        pltpu.sync_copy(
            deg_sh.at[pl.ds(s * DEG_PER_SUB, DEG_PER_SUB)],
            deg_out.at[c, pl.ds(s * DEG_PER_SUB, DEG_PER_SUB)],
        )

    return k(h, src, dst)


def _pre_body(x_ref, w_ref, b_ref, o_ref):
    x = x_ref[...]
    n = jnp.sqrt(jnp.sum(x * x, axis=1, keepdims=True))
    n = jnp.maximum(n, 1e-7)
    u = (jnp.arctan2(n, jnp.ones_like(n)) / n) * x
    o_ref[...] = (
        lax.dot(u, w_ref[...], preferred_element_type=jnp.float32) + b_ref[...]
    )


def _combine(h_ref, a_ref, d_ref):
    h = h_ref[...]
    agg = a_ref[0] + a_ref[1]
    rows = pl.ds(pl.program_id(0) * ROWS_BLK, ROWS_BLK)
    deg = jnp.sum(d_ref[:, rows], axis=0)
    agg = agg / jnp.maximum(deg, 1.0)[:, None]
    out = jnp.maximum(h + agg, 0.0)
    nrm = jnp.sqrt(jnp.sum(out * out, axis=1, keepdims=True))
    return out / (nrm + 1e-7)


def _mid_body(h_ref, a_ref, d_ref, w_ref, b_ref, o_ref):
    u = _combine(h_ref, a_ref, d_ref)
    o_ref[...] = (
        lax.dot(u, w_ref[...], preferred_element_type=jnp.float32) + b_ref[...]
    )


def _post_body(h_ref, a_ref, d_ref, o_ref):
    u = _combine(h_ref, a_ref, d_ref)
    n = jnp.sqrt(jnp.sum(u * u, axis=1, keepdims=True))
    n = jnp.maximum(n, 1e-7)
    t = jnp.clip(n, 0.0, 1.5)
    o_ref[...] = (jnp.tan(t) / n) * u


def _row_grid(nrows):
    return pl.cdiv(nrows, ROWS_BLK)


_W_SPEC = pl.BlockSpec((D, D), lambda i: (0, 0))
_B_SPEC = pl.BlockSpec((1, D), lambda i: (0, 0))
_ROW_SPEC = pl.BlockSpec((ROWS_BLK, D), lambda i: (i, 0))
_AGG_SPEC = pl.BlockSpec((NC, ROWS_BLK, D), lambda i: (0, i, 0))
_DEG_SPEC = pl.BlockSpec((NC, NPAD), lambda i: (0, 0))


def _pre(x, w, b):
    return pl.pallas_call(
        _pre_body,
        grid=(_row_grid(N),),
        in_specs=[_ROW_SPEC, _W_SPEC, _B_SPEC],
        out_specs=_ROW_SPEC,
        out_shape=jax.ShapeDtypeStruct((N, D), jnp.float32),
    )(x, w, b)


def _mid(h, agg_part, deg_part, w, b):
    return pl.pallas_call(
        _mid_body,
        grid=(_row_grid(N),),
        in_specs=[_ROW_SPEC, _AGG_SPEC, _DEG_SPEC, _W_SPEC, _B_SPEC],
        out_specs=_ROW_SPEC,
        out_shape=jax.ShapeDtypeStruct((N, D), jnp.float32),
    )(h, agg_part, deg_part, w, b)


def _post(h, agg_part, deg_part):
    return pl.pallas_call(
        _post_body,
        grid=(_row_grid(N),),
        in_specs=[_ROW_SPEC, _AGG_SPEC, _DEG_SPEC],
        out_specs=_ROW_SPEC,
        out_shape=jax.ShapeDtypeStruct((N, D), jnp.float32),
    )(h, agg_part, deg_part)


def kernel(x, adj, W1, b1, W2, b2):
    b1r = b1.reshape(1, D)
    b2r = b2.reshape(1, D)

    src1 = adj[0, 0].reshape(NW, NCHUNK, CHUNK)
    dst1 = adj[0, 1].reshape(NW, NCHUNK, CHUNK)
    src2 = adj[1, 0].reshape(NW, NCHUNK, CHUNK)
    dst2 = adj[1, 1].reshape(NW, NCHUNK, CHUNK)

    h1 = _pre(x, W1, b1r)
    agg1, deg1 = _sc_aggregate(h1, src1, dst1)
    h2 = _mid(h1, agg1, deg1, W2, b2r)
    agg2, deg2 = _sc_aggregate(h2, src2, dst2)
    return _post(h2, agg2, deg2)


# speedup vs baseline: 1.5405x; 1.5405x over previous
import functools

import jax
import jax.numpy as jnp
from jax import lax
from jax.experimental import pallas as pl
from jax.experimental.pallas import tpu as pltpu
from jax.experimental.pallas import tpu_sc as plsc

N = 10000
D = 128
E = 320000

NC = 2
NS = 16
NW = NC * NS
EPW = E // NW
CHUNK = 80
NCHUNK = EPW // CHUNK
NPAD = 10240
AGG_PER_SUB = NPAD // NS
DEG_PER_SUB = NPAD // NS

ROWS_BLK = 1024


def _sc_aggregate(h, src, dst):
    mesh = plsc.VectorSubcoreMesh(
        core_axis_name="c", subcore_axis_name="s", num_cores=NC, num_subcores=NS
    )

    @functools.partial(
        pl.kernel,
        out_type=(
            jax.ShapeDtypeStruct((NC, NPAD, D), jnp.float32),
            jax.ShapeDtypeStruct((NC, NPAD), jnp.float32),
        ),
        mesh=mesh,
        scratch_types=(
            pltpu.VMEM_SHARED((NPAD, D), jnp.float32),
            pltpu.VMEM_SHARED((NPAD,), jnp.float32),
            pltpu.VMEM((CHUNK,), jnp.int32),
            pltpu.VMEM((CHUNK,), jnp.int32),
            pltpu.VMEM((CHUNK,), jnp.int32),
            pltpu.VMEM((CHUNK,), jnp.int32),
            pltpu.VMEM((CHUNK, D), jnp.float32),
            pltpu.VMEM((CHUNK, D), jnp.float32),
            pltpu.VMEM((CHUNK,), jnp.float32),
            pltpu.VMEM((DEG_PER_SUB,), jnp.float32),
            pltpu.SemaphoreType.DMA,
            pltpu.SemaphoreType.DMA,
        ),
    )
    def k(h_hbm, src_hbm, dst_hbm,
          agg_out, deg_out, agg_sh, deg_sh, sa_s, sa_d, sb_s, sb_d,
          rows0, rows1, ones_v, zcol, g0, g1):
        c = lax.axis_index("c")
        s = lax.axis_index("s")
        wid = c * NS + s

        o16 = jnp.ones((16,), jnp.float32)
        z16 = jnp.zeros((16,), jnp.float32)
        for t in range(CHUNK // 16):
            ones_v[pl.ds(16 * t, 16)] = o16

        def zrow(i, carry):
            r = i // (D // 16)
            q = i - r * (D // 16)
            rows0[r, pl.ds(q * 16, 16)] = z16
            return carry

        lax.fori_loop(0, CHUNK * (D // 16), zrow, 0)

        def zvec(i, carry):
            zcol[pl.ds(i * 16, 16)] = z16
            return carry

        lax.fori_loop(0, DEG_PER_SUB // 16, zvec, 0)

        for t in range(AGG_PER_SUB // CHUNK):
            pltpu.sync_copy(
                rows0, agg_sh.at[pl.ds(s * AGG_PER_SUB + t * CHUNK, CHUNK)]
            )
        pltpu.sync_copy(zcol, deg_sh.at[pl.ds(s * DEG_PER_SUB, DEG_PER_SUB)])
        plsc.subcore_barrier()

        def idx_fetch(ci, bs, bd):
            pltpu.sync_copy(src_hbm.at[wid, ci], bs)
            pltpu.sync_copy(dst_hbm.at[wid, ci], bd)

        def g_fire(sidx, rows, sem):
            pltpu.async_copy(h_hbm.at[sidx], rows, sem)

        def g_wait(rows, sem):
            pltpu.make_async_copy(h_hbm.at[sa_s], rows, sem).wait()

        def scat(didx, rows):
            pltpu.sync_copy(rows, agg_sh.at[didx], add=True)
            pltpu.sync_copy(ones_v, deg_sh.at[didx], add=True)

        idx_fetch(0, sa_s, sa_d)
        g_fire(sa_s, rows0, g0)

        def body(i, carry):
            c0 = 2 * i
            idx_fetch(c0 + 1, sb_s, sb_d)
            g_fire(sb_s, rows1, g1)
            g_wait(rows0, g0)
            scat(sa_d, rows0)
            idx_fetch(c0 + 2, sa_s, sa_d)
            g_fire(sa_s, rows0, g0)
            g_wait(rows1, g1)
            scat(sb_d, rows1)
            return carry

        lax.fori_loop(0, (NCHUNK - 1) // 2, body, 0)
        g_wait(rows0, g0)
        scat(sa_d, rows0)
        plsc.subcore_barrier()

        pltpu.sync_copy(
            agg_sh.at[pl.ds(s * AGG_PER_SUB, AGG_PER_SUB)],
            agg_out.at[c, pl.ds(s * AGG_PER_SUB, AGG_PER_SUB)],
        )
        pltpu.sync_copy(
            deg_sh.at[pl.ds(s * DEG_PER_SUB, DEG_PER_SUB)],
            deg_out.at[c, pl.ds(s * DEG_PER_SUB, DEG_PER_SUB)],
        )

    return k(h, src, dst)


def _pre_body(x_ref, w_ref, b_ref, o_ref):
    x = x_ref[...]
    n = jnp.sqrt(jnp.sum(x * x, axis=1, keepdims=True))
    n = jnp.maximum(n, 1e-7)
    u = (jnp.arctan2(n, jnp.ones_like(n)) / n) * x
    o_ref[...] = (
        lax.dot(u, w_ref[...], preferred_element_type=jnp.float32) + b_ref[...]
    )


def _combine(h_ref, a_ref, d_ref):
    h = h_ref[...]
    agg = a_ref[0] + a_ref[1]
    rows = pl.ds(pl.program_id(0) * ROWS_BLK, ROWS_BLK)
    deg = jnp.sum(d_ref[:, rows], axis=0)
    agg = agg / jnp.maximum(deg, 1.0)[:, None]
    out = jnp.maximum(h + agg, 0.0)
    nrm = jnp.sqrt(jnp.sum(out * out, axis=1, keepdims=True))
    return out / (nrm + 1e-7)


def _mid_body(h_ref, a_ref, d_ref, w_ref, b_ref, o_ref):
    u = _combine(h_ref, a_ref, d_ref)
    o_ref[...] = (
        lax.dot(u, w_ref[...], preferred_element_type=jnp.float32) + b_ref[...]
    )


def _post_body(h_ref, a_ref, d_ref, o_ref):
    u = _combine(h_ref, a_ref, d_ref)
    n = jnp.sqrt(jnp.sum(u * u, axis=1, keepdims=True))
    n = jnp.maximum(n, 1e-7)
    t = jnp.clip(n, 0.0, 1.5)
    o_ref[...] = (jnp.tan(t) / n) * u


def _row_grid(nrows):
    return pl.cdiv(nrows, ROWS_BLK)


_W_SPEC = pl.BlockSpec((D, D), lambda i: (0, 0))
_B_SPEC = pl.BlockSpec((1, D), lambda i: (0, 0))
_ROW_SPEC = pl.BlockSpec((ROWS_BLK, D), lambda i: (i, 0))
_AGG_SPEC = pl.BlockSpec((NC, ROWS_BLK, D), lambda i: (0, i, 0))
_DEG_SPEC = pl.BlockSpec((NC, NPAD), lambda i: (0, 0))


def _pre(x, w, b):
    return pl.pallas_call(
        _pre_body,
        grid=(_row_grid(N),),
        in_specs=[_ROW_SPEC, _W_SPEC, _B_SPEC],
        out_specs=_ROW_SPEC,
        out_shape=jax.ShapeDtypeStruct((N, D), jnp.float32),
    )(x, w, b)


def _mid(h, agg_part, deg_part, w, b):
    return pl.pallas_call(
        _mid_body,
        grid=(_row_grid(N),),
        in_specs=[_ROW_SPEC, _AGG_SPEC, _DEG_SPEC, _W_SPEC, _B_SPEC],
        out_specs=_ROW_SPEC,
        out_shape=jax.ShapeDtypeStruct((N, D), jnp.float32),
    )(h, agg_part, deg_part, w, b)


def _post(h, agg_part, deg_part):
    return pl.pallas_call(
        _post_body,
        grid=(_row_grid(N),),
        in_specs=[_ROW_SPEC, _AGG_SPEC, _DEG_SPEC],
        out_specs=_ROW_SPEC,
        out_shape=jax.ShapeDtypeStruct((N, D), jnp.float32),
    )(h, agg_part, deg_part)


def kernel(x, adj, W1, b1, W2, b2):
    b1r = b1.reshape(1, D)
    b2r = b2.reshape(1, D)

    src1 = adj[0, 0].reshape(NW, NCHUNK, CHUNK)
    dst1 = adj[0, 1].reshape(NW, NCHUNK, CHUNK)
    src2 = adj[1, 0].reshape(NW, NCHUNK, CHUNK)
    dst2 = adj[1, 1].reshape(NW, NCHUNK, CHUNK)

    h1 = _pre(x, W1, b1r)
    agg1, deg1 = _sc_aggregate(h1, src1, dst1)
    h2 = _mid(h1, agg1, deg1, W2, b2r)
    agg2, deg2 = _sc_aggregate(h2, src2, dst2)
    return _post(h2, agg2, deg2)
